# baseline (speedup 1.0000x reference)
import jax
import jax.numpy as jnp
from jax.experimental import pallas as pl
from jax.experimental.pallas import tpu as pltpu

_B, _C, _L = 32, 64, 8192
_K = 4
_OC = _C * _K
_LOUT = 4099
_NCH = 32


def _sp_kernel(x_ref, t_ref, w_ref, o_ref, zs_ref):
    x = x_ref[0].astype(jnp.bfloat16)
    tall = t_ref[...]
    z256 = jnp.zeros((_C, 256), dtype=jnp.bfloat16)
    for m in range(_NCH + 1):
        if m == 0:
            v = jnp.concatenate([z256, x[:, :256]], axis=1)
        elif m == _NCH:
            v = jnp.concatenate([x[:, _L - 256:], z256], axis=1)
        else:
            v = x[:, 256 * (m - 1):256 * (m + 1)]
        z = jax.lax.dot_general(
            v, tall, (((1,), (0,)), ((), ())),
            preferred_element_type=jnp.float32)
        zb = z.astype(jnp.bfloat16)
        for k in range(_K):
            zs_ref[64 * k:64 * (k + 1), 128 * m:128 * (m + 1)] = (
                zb[:, 128 * k:128 * (k + 1)])
    w = w_ref[...]
    for m in range(_NCH + 1):
        o = jax.lax.dot_general(
            w, zs_ref[:, 128 * m:128 * (m + 1)], (((1,), (0,)), ((), ())),
            preferred_element_type=jnp.float32)
        o_ref[0, :, m * 128:(m + 1) * 128] = o


def kernel(x, weight):
    cols = jnp.arange(512)
    qsel = 250 + 2 * (cols % 128) + 2 * (cols // 128)
    tall = (jnp.arange(512)[:, None] == qsel[None, :]).astype(jnp.bfloat16)
    wbig = jnp.transpose(weight, (0, 2, 1)).reshape(_OC, _OC).astype(
        jnp.bfloat16)
    padded = pl.pallas_call(
        _sp_kernel,
        grid=(_B,),
        in_specs=[
            pl.BlockSpec((1, _C, _L), lambda b: (b, 0, 0)),
            pl.BlockSpec((512, 512), lambda b: (0, 0)),
            pl.BlockSpec((_OC, _OC), lambda b: (0, 0)),
        ],
        out_specs=pl.BlockSpec((1, _OC, 128 * (_NCH + 1)), lambda b: (b, 0, 0)),
        out_shape=jax.ShapeDtypeStruct((_B, _OC, 128 * (_NCH + 1)), jnp.float32),
        scratch_shapes=[pltpu.VMEM((_OC, 128 * (_NCH + 1)), jnp.bfloat16)],
        compiler_params=pltpu.CompilerParams(
            dimension_semantics=("parallel",),
            vmem_limit_bytes=100 * 1024 * 1024,
        ),
    )(x, tall, wbig)
    return padded[:, :, :_LOUT]
